# Initial kernel scaffold; baseline (speedup 1.0000x reference)
import jax
import jax.numpy as jnp
from jax.experimental import pallas as pl


def kernel(pc, flow):
    raise NotImplementedError("write your pallas kernel here")


# speedup vs baseline: 14.8025x; 14.8025x over previous
import functools

import jax
import jax.numpy as jnp
from jax.experimental import pallas as pl
from jax.experimental.pallas import tpu as pltpu

_K = 16
_RADIUS2 = 0.0625
_BISECT_ITERS = 24
_ROW_BLOCK = 128


def _knn_loss_block(pc_blk_ref, pcT_ref, flow_blk_ref, flowT_ref, out_ref):
    b = pl.program_id(0)
    i = pl.program_id(1)

    pc_blk = pc_blk_ref[0]
    pcT = pcT_ref[0]
    flow_blk = flow_blk_ref[0]
    flowT = flowT_ref[0]

    pb = pc_blk.astype(jnp.bfloat16).astype(jnp.float32)
    pt = pcT.astype(jnp.bfloat16).astype(jnp.float32)
    dot = (pb[:, 0:1] * pt[0:1, :]
           + pb[:, 1:2] * pt[1:2, :]
           + pb[:, 2:3] * pt[2:3, :])
    sq_r = jnp.sum(pc_blk * pc_blk, axis=1, keepdims=True)
    sq_c = jnp.sum(pcT * pcT, axis=0, keepdims=True)
    d2 = jnp.maximum(sq_r + sq_c - 2.0 * dot, 0.0)

    kf = jnp.float32(_K)

    c_hi0 = jnp.sum((d2 <= _RADIUS2).astype(jnp.float32), axis=1,
                    keepdims=True)
    lo0 = jnp.full_like(c_hi0, -1.0)
    hi0 = jnp.full_like(c_hi0, _RADIUS2)
    c_lo0 = jnp.zeros_like(c_hi0)

    def body(_, st):
        lo, hi, c_lo, c_hi = st
        mid = 0.5 * (lo + hi)
        cnt = jnp.sum((d2 <= mid).astype(jnp.float32), axis=1, keepdims=True)
        pred = cnt >= kf
        lo_n = jnp.where(pred, lo, mid)
        c_lo_n = jnp.where(pred, c_lo, cnt)
        hi_n = jnp.where(pred, mid, hi)
        c_hi_n = jnp.where(pred, cnt, c_hi)
        return lo_n, hi_n, c_lo_n, c_hi_n

    lo, hi, c_lo, c_hi = jax.lax.fori_loop(
        0, _BISECT_ITERS, body, (lo0, hi0, c_lo0, c_hi0))

    l1 = (jnp.abs(flow_blk[:, 0:1] - flowT[0:1, :])
          + jnp.abs(flow_blk[:, 1:2] - flowT[1:2, :])
          + jnp.abs(flow_blk[:, 2:3] - flowT[2:3, :]))

    s_lo = jnp.sum(jnp.where(d2 <= lo, l1, 0.0), axis=1, keepdims=True)
    s_hi = jnp.sum(jnp.where(d2 <= hi, l1, 0.0), axis=1, keepdims=True)

    denom = jnp.maximum(c_hi - c_lo, 1.0)
    sel = jnp.where(c_hi <= kf,
                    s_hi,
                    s_lo + (kf - c_lo) * (s_hi - s_lo) / denom)

    n_cols = d2.shape[1]
    rowmin = jnp.min(d2, axis=1, keepdims=True)
    iota = jax.lax.broadcasted_iota(jnp.int32, d2.shape, 1)
    cand = jnp.where(d2 == rowmin, iota, jnp.int32(n_cols))
    amin = jnp.min(cand, axis=1, keepdims=True)
    l1min = jnp.sum(jnp.where(iota == amin, l1, 0.0), axis=1, keepdims=True)
    repl = jnp.maximum(kf - c_hi0, 0.0)
    sel = sel + repl * l1min

    part = jnp.sum(sel).reshape(1, 1)

    @pl.when(jnp.logical_and(b == 0, i == 0))
    def _init():
        out_ref[...] = jnp.zeros_like(out_ref)

    out_ref[...] += part


def kernel(pc, flow):
    B, N, _ = pc.shape
    rb = _ROW_BLOCK
    pcT = jnp.transpose(pc, (0, 2, 1))
    flowT = jnp.transpose(flow, (0, 2, 1))

    grid = (B, N // rb)
    total = pl.pallas_call(
        _knn_loss_block,
        grid=grid,
        in_specs=[
            pl.BlockSpec((1, rb, 3), lambda b, i: (b, i, 0)),
            pl.BlockSpec((1, 3, N), lambda b, i: (b, 0, 0)),
            pl.BlockSpec((1, rb, 3), lambda b, i: (b, i, 0)),
            pl.BlockSpec((1, 3, N), lambda b, i: (b, 0, 0)),
        ],
        out_specs=pl.BlockSpec((1, 1), lambda b, i: (0, 0)),
        out_shape=jax.ShapeDtypeStruct((1, 1), jnp.float32),
    )(pc, pcT, flow, flowT)

    return total[0, 0] / jnp.float32(B * N * _K)


# speedup vs baseline: 19.3150x; 1.3048x over previous
import functools

import jax
import jax.numpy as jnp
from jax.experimental import pallas as pl
from jax.experimental.pallas import tpu as pltpu

_K = 16
_RADIUS2 = 0.0625
_SEARCH_STEPS = 7
_ROW_BLOCK = 128


def _knn_loss_block(pc_blk_ref, pcT_ref, flow_blk_ref, flowT_ref, out_ref):
    b = pl.program_id(0)
    i = pl.program_id(1)

    pc_blk = pc_blk_ref[0]
    pcT = pcT_ref[0]
    flow_blk = flow_blk_ref[0]
    flowT = flowT_ref[0]

    pb = pc_blk.astype(jnp.bfloat16).astype(jnp.float32)
    pt = pcT.astype(jnp.bfloat16).astype(jnp.float32)
    dot = (pb[:, 0:1] * pt[0:1, :]
           + pb[:, 1:2] * pt[1:2, :]
           + pb[:, 2:3] * pt[2:3, :])
    sq_r = jnp.sum(pc_blk * pc_blk, axis=1, keepdims=True)
    sq_c = jnp.sum(pcT * pcT, axis=0, keepdims=True)
    d2 = jnp.maximum(sq_r + sq_c - 2.0 * dot, 0.0)

    kf = jnp.float32(_K)

    c_hi0 = jnp.sum((d2 <= _RADIUS2).astype(jnp.float32), axis=1,
                    keepdims=True)
    rowmin = jnp.min(d2, axis=1, keepdims=True)
    lo0 = jnp.full_like(c_hi0, -1.0)
    hi0 = jnp.full_like(c_hi0, _RADIUS2)
    c_lo0 = jnp.zeros_like(c_hi0)

    def body(_, st):
        lo, hi, c_lo, c_hi = st
        w = hi - lo
        m1 = lo + 0.25 * w
        m2 = lo + 0.5 * w
        m3 = lo + 0.75 * w
        cm1 = (d2 <= m1).astype(jnp.float32)
        cm2 = (d2 <= m2).astype(jnp.float32)
        cm3 = (d2 <= m3).astype(jnp.float32)
        c1 = jnp.sum(cm1, axis=1, keepdims=True)
        c2 = jnp.sum(cm2, axis=1, keepdims=True)
        c3 = jnp.sum(cm3, axis=1, keepdims=True)
        p1 = c1 >= kf
        p2 = c2 >= kf
        p3 = c3 >= kf
        hi_n = jnp.where(p1, m1, jnp.where(p2, m2, jnp.where(p3, m3, hi)))
        c_hi_n = jnp.where(p1, c1, jnp.where(p2, c2, jnp.where(p3, c3, c_hi)))
        lo_n = jnp.where(p1, lo, jnp.where(p2, m1, jnp.where(p3, m2, m3)))
        c_lo_n = jnp.where(p1, c_lo,
                           jnp.where(p2, c1, jnp.where(p3, c2, c3)))
        return lo_n, hi_n, c_lo_n, c_hi_n

    lo, hi, c_lo, c_hi = jax.lax.fori_loop(
        0, _SEARCH_STEPS, body, (lo0, hi0, c_lo0, c_hi0))

    l1 = (jnp.abs(flow_blk[:, 0:1] - flowT[0:1, :])
          + jnp.abs(flow_blk[:, 1:2] - flowT[1:2, :])
          + jnp.abs(flow_blk[:, 2:3] - flowT[2:3, :]))

    in_lo = d2 <= lo
    in_hi = d2 <= hi
    s_lo = jnp.sum(jnp.where(in_lo, l1, 0.0), axis=1, keepdims=True)
    s_hi = jnp.sum(jnp.where(in_hi, l1, 0.0), axis=1, keepdims=True)

    denom = jnp.maximum(c_hi - c_lo, 1.0)
    sel = jnp.where(c_hi <= kf,
                    s_hi,
                    s_lo + (kf - c_lo) * (s_hi - s_lo) / denom)

    n_cols = d2.shape[1]
    iota = jax.lax.broadcasted_iota(jnp.int32, d2.shape, 1)
    cand = jnp.where(d2 == rowmin, iota, jnp.int32(n_cols))
    amin = jnp.min(cand, axis=1, keepdims=True)
    l1min = jnp.sum(jnp.where(iota == amin, l1, 0.0), axis=1, keepdims=True)
    repl = jnp.maximum(kf - c_hi0, 0.0)
    sel = sel + repl * l1min

    part = jnp.sum(sel).reshape(1, 1)

    @pl.when(jnp.logical_and(b == 0, i == 0))
    def _init():
        out_ref[...] = jnp.zeros_like(out_ref)

    out_ref[...] += part


def kernel(pc, flow):
    B, N, _ = pc.shape
    rb = _ROW_BLOCK
    pcT = jnp.transpose(pc, (0, 2, 1))
    flowT = jnp.transpose(flow, (0, 2, 1))

    grid = (B, N // rb)
    total = pl.pallas_call(
        _knn_loss_block,
        grid=grid,
        in_specs=[
            pl.BlockSpec((1, rb, 3), lambda b, i: (b, i, 0)),
            pl.BlockSpec((1, 3, N), lambda b, i: (b, 0, 0)),
            pl.BlockSpec((1, rb, 3), lambda b, i: (b, i, 0)),
            pl.BlockSpec((1, 3, N), lambda b, i: (b, 0, 0)),
        ],
        out_specs=pl.BlockSpec((1, 1), lambda b, i: (0, 0)),
        out_shape=jax.ShapeDtypeStruct((1, 1), jnp.float32),
    )(pc, pcT, flow, flowT)

    return total[0, 0] / jnp.float32(B * N * _K)
